# Initial kernel scaffold; baseline (speedup 1.0000x reference)
import jax
import jax.numpy as jnp
from jax.experimental import pallas as pl


def kernel(x):
    raise NotImplementedError("write your pallas kernel here")


# speedup vs baseline: 2.0240x; 2.0240x over previous
import functools

import jax
import jax.numpy as jnp
from jax.experimental import pallas as pl
from jax.experimental.pallas import tpu as pltpu

N = 2048
H = 128
W = 128
M0 = 0.01
BOUND = M0 * N
K = 21
BIG = 3.4e38


def _dtm_kernel(x_ref, out_ref, d2_ref):
    i = pl.program_id(0)
    gy = 1.0 - i.astype(jnp.float32) * (2.0 / (W - 1))
    gx = -1.0 + jax.lax.broadcasted_iota(
        jnp.int32, (1, W), 1).astype(jnp.float32) * (2.0 / (W - 1))

    px = x_ref[:, 0:1]
    py = x_ref[:, 1:2]

    dx = px - gx
    dy = py - gy
    d2_ref[...] = dx * dx + dy * dy

    def body(_, carry):
        s, t, rem = carry
        d = d2_ref[...]
        m = jnp.min(d, axis=0, keepdims=True)
        mask = d == m
        c = jnp.sum(mask.astype(jnp.float32), axis=0, keepdims=True)
        d2_ref[...] = jnp.where(mask, BIG, d)
        take = jnp.minimum(c, rem)
        s = s + take * m
        t = jnp.where((rem > 0.0) & (rem <= take), m, t)
        rem = rem - take
        return s, t, rem

    zero = jnp.zeros((1, W), jnp.float32)
    s, t, _ = jax.lax.fori_loop(
        0, K, body, (zero, zero, jnp.full((1, W), float(K), jnp.float32)))

    dtm_val = s + t * (BOUND - K)
    out_ref[0] = jnp.sqrt(dtm_val / BOUND)


@jax.jit
def kernel(x):
    out = pl.pallas_call(
        _dtm_kernel,
        grid=(H,),
        in_specs=[pl.BlockSpec((N, 2), lambda i: (0, 0))],
        out_specs=pl.BlockSpec((1, 1, W), lambda i: (i, 0, 0)),
        out_shape=jax.ShapeDtypeStruct((H, 1, W), jnp.float32),
        scratch_shapes=[pltpu.VMEM((N, W), jnp.float32)],
    )(x)
    return out.reshape(H, W)
